# baseline (speedup 1.0000x reference)
import functools

import jax
import jax.numpy as jnp
from jax import lax
from jax.experimental import pallas as pl
from jax.experimental.pallas import tpu as pltpu
from jax.experimental.pallas import tpu_sc as plsc

_B = 16384
_D = 32
_H = 64
_NC = 2
_NS = 16
_NW = _NC * _NS
_BPW = _B // _NW

@functools.cache
def _sc_gather_fn():
    mesh = plsc.VectorSubcoreMesh(
        core_axis_name="c", subcore_axis_name="s",
        num_cores=_NC, num_subcores=_NS,
    )

    @functools.partial(
        pl.kernel,
        out_type=(
            jax.ShapeDtypeStruct((_B, _D), jnp.float32),
            jax.ShapeDtypeStruct((_B, _D), jnp.float32),
        ),
        mesh=mesh,
        compiler_params=pltpu.CompilerParams(use_tc_tiling_on_sc=False),
        scratch_types=[
            pltpu.VMEM((_BPW,), jnp.int32),
            pltpu.VMEM((_BPW,), jnp.int32),
            pltpu.VMEM((_BPW, _D), jnp.float32),
            pltpu.VMEM((_BPW, _D), jnp.float32),
            pltpu.SemaphoreType.DMA,
            pltpu.SemaphoreType.DMA,
        ],
    )
    def sc_gather(U_hbm, V_hbm, u_hbm, i_hbm, eu_hbm, ev_hbm,
                  uidx, iidx, eu_v, ev_v, sem_u, sem_v):
        wid = lax.axis_index("s") * _NC + lax.axis_index("c")
        base = wid * _BPW
        pltpu.sync_copy(u_hbm.at[pl.ds(base, _BPW)], uidx)
        pltpu.sync_copy(i_hbm.at[pl.ds(base, _BPW)], iidx)
        cu = pltpu.async_copy(U_hbm.at[uidx], eu_v, sem_u)
        cv = pltpu.async_copy(V_hbm.at[iidx], ev_v, sem_v)
        cu.wait()
        pltpu.sync_copy(eu_v, eu_hbm.at[pl.ds(base, _BPW)])
        cv.wait()
        pltpu.sync_copy(ev_v, ev_hbm.at[pl.ds(base, _BPW)])

    return sc_gather


_BLK = 2048


def _mlp_body(eu_ref, ev_ref, w1a_ref, w1b_ref, b1_ref, w2_ref, b2_ref, o_ref):
    h = jnp.dot(eu_ref[...], w1a_ref[...], preferred_element_type=jnp.float32)
    h = h + jnp.dot(ev_ref[...], w1b_ref[...], preferred_element_type=jnp.float32)
    h = jnp.maximum(h + b1_ref[...], 0.0)
    z = jnp.sum(h * w2_ref[...], axis=1) + b2_ref[0, 0]
    o_ref[...] = 1.0 / (1.0 + jnp.exp(-z))


_mlp = pl.pallas_call(
    _mlp_body,
    grid=(_B // _BLK,),
    in_specs=[
        pl.BlockSpec((_BLK, _D), lambda j: (j, 0)),
        pl.BlockSpec((_BLK, _D), lambda j: (j, 0)),
        pl.BlockSpec((_D, _H), lambda j: (0, 0)),
        pl.BlockSpec((_D, _H), lambda j: (0, 0)),
        pl.BlockSpec((1, _H), lambda j: (0, 0)),
        pl.BlockSpec((1, _H), lambda j: (0, 0)),
        pl.BlockSpec((1, 1), lambda j: (0, 0)),
    ],
    out_specs=pl.BlockSpec((_BLK,), lambda j: (j,)),
    out_shape=jax.ShapeDtypeStruct((_B,), jnp.float32),
)


def kernel(u, i, U, V, W1, b1, W2, b2):
    u = u.astype(jnp.int32)
    i = i.astype(jnp.int32)
    eu, ev = _sc_gather_fn()(U, V, u, i)
    w1a = W1[:, :_D].T
    w1b = W1[:, _D:].T
    return _mlp(eu, ev, w1a, w1b,
                b1.reshape(1, _H), W2, b2.reshape(1, 1))
